# Initial kernel scaffold; baseline (speedup 1.0000x reference)
import jax
import jax.numpy as jnp
from jax.experimental import pallas as pl


def kernel(inputs, embedding):
    raise NotImplementedError("write your pallas kernel here")


# speedup vs baseline: 1.4018x; 1.4018x over previous
import jax
import jax.numpy as jnp
from jax.experimental import pallas as pl
from jax.experimental.pallas import tpu as pltpu

SOM_H = 32
SOM_W = 32
K = SOM_H * SOM_W
C = 256
ALPHA = 6.0
BETA = 1.0
V = 512
N = 2 * 8 * 32 * 32
M = 8 * 32 * 32


def _body(x_ref, et_ref, en_ref, enc_ref, qt_ref, scal_ref):
    step = pl.program_id(0)
    A = x_ref[0]
    ET = et_ref[...]
    en = en_ref[...]

    prod = jax.lax.dot_general(
        A, ET, (((0,), (0,)), ((), ())),
        preferred_element_type=jnp.float32,
        precision=jax.lax.Precision.DEFAULT)
    P = en - 2.0 * prod

    m = jnp.min(P, axis=1, keepdims=True)
    iota_k = jax.lax.broadcasted_iota(jnp.int32, (V, K), 1)
    idx = jnp.min(jnp.where(P == m, iota_k, jnp.int32(K)), axis=1)

    enc_ref[...] = (iota_k == idx[:, None]).astype(jnp.float32)

    encT = (jax.lax.broadcasted_iota(jnp.int32, (K, V), 0)
            == idx[None, :]).astype(jnp.float32)
    qt_ref[0] = jax.lax.dot_general(
        ET, encT, (((1,), (0,)), ((), ())),
        preferred_element_type=jnp.float32,
        precision=jax.lax.Precision.HIGHEST)

    xn = jnp.sum(A * A, axis=0)
    jj = idx % SOM_W
    ii = idx // SOM_W
    f32 = jnp.float32
    lmask = (iota_k == (idx - 1)[:, None]) & (jj > 0)[:, None]
    rmask = (iota_k == (idx + 1)[:, None]) & (jj < SOM_W - 1)[:, None]
    umask = iota_k == (idx - SOM_W)[:, None]
    dmask = iota_k == (idx + SOM_W)[:, None]
    nbr = lmask | rmask | umask | dmask
    nv = (1.0 + (jj > 0).astype(f32) + (jj < SOM_W - 1).astype(f32)
          + (ii > 0).astype(f32) + (ii < SOM_H - 1).astype(f32))

    vals = (
        jnp.sum(m),
        jnp.sum(xn),
        jnp.sum(jnp.where(nbr, P, 0.0)) + jnp.sum(m),
        jnp.sum(nv * xn),
        jnp.sum(nv),
    )
    for k, v in enumerate(vals):
        prev = jnp.where(step == 0, 0.0, scal_ref[k])
        scal_ref[k] = prev + v


def kernel(inputs, embedding):
    x3 = inputs.reshape(2, C, M)
    et = embedding.T
    en = jnp.sum(embedding * embedding, axis=1)[None, :]

    enc, qt, s = pl.pallas_call(
        _body,
        grid=(N // V,),
        in_specs=[
            pl.BlockSpec((1, C, V), lambda i: (i // (M // V), 0, i % (M // V))),
            pl.BlockSpec((C, K), lambda i: (0, 0)),
            pl.BlockSpec((1, K), lambda i: (0, 0)),
        ],
        out_specs=(
            pl.BlockSpec((V, K), lambda i: (i, 0)),
            pl.BlockSpec((1, C, V), lambda i: (i // (M // V), 0, i % (M // V))),
            pl.BlockSpec(memory_space=pltpu.SMEM),
        ),
        out_shape=(
            jax.ShapeDtypeStruct((N, K), jnp.float32),
            jax.ShapeDtypeStruct((2, C, M), jnp.float32),
            jax.ShapeDtypeStruct((8,), jnp.float32),
        ),
    )(x3, et, en)

    commitment = (s[0] + s[1]) / (N * C)
    somloss = (s[2] + s[3]) / s[4]
    loss = ALPHA * commitment + BETA * somloss
    return (loss, qt.reshape(2, C, 8, 32, 32), enc)


# speedup vs baseline: 1.5910x; 1.1350x over previous
import jax
import jax.numpy as jnp
from jax.experimental import pallas as pl
from jax.experimental.pallas import tpu as pltpu

SOM_H = 32
SOM_W = 32
K = SOM_H * SOM_W
C = 256
ALPHA = 6.0
BETA = 1.0
V = 512
N = 2 * 8 * 32 * 32
M = 8 * 32 * 32


def _body(x_ref, et_ref, en_ref, ehi_ref, elo_ref, enc_ref, qt_ref, scal_ref):
    step = pl.program_id(0)
    A = x_ref[0]
    ET = et_ref[...]
    en = en_ref[...]

    prod = jax.lax.dot_general(
        A, ET, (((0,), (0,)), ((), ())),
        preferred_element_type=jnp.float32,
        precision=jax.lax.Precision.DEFAULT)
    P = en - 2.0 * prod

    m = jnp.min(P, axis=1, keepdims=True)
    iota_k = jax.lax.broadcasted_iota(jnp.int32, (V, K), 1)
    idx = jnp.min(jnp.where(P == m, iota_k, jnp.int32(K)), axis=1)

    enc = (iota_k == idx[:, None]).astype(jnp.float32)
    enc_ref[...] = enc

    encT = (jax.lax.broadcasted_iota(jnp.int32, (K, V), 0)
            == idx[None, :]).astype(jnp.bfloat16)
    qt_ref[0] = (
        jax.lax.dot_general(ehi_ref[...], encT, (((1,), (0,)), ((), ())),
                            preferred_element_type=jnp.float32)
        + jax.lax.dot_general(elo_ref[...], encT, (((1,), (0,)), ((), ())),
                              preferred_element_type=jnp.float32))

    xn = jnp.sum(A * A, axis=0)
    jj = idx % SOM_W
    ii = idx // SOM_W
    f32 = jnp.float32
    z1 = jnp.zeros((V, 1), f32)
    z32 = jnp.zeros((V, SOM_W), f32)
    sr1 = jnp.concatenate([z1, P[:, :-1]], axis=1)
    sl1 = jnp.concatenate([P[:, 1:], z1], axis=1)
    sr32 = jnp.concatenate([z32, P[:, :-SOM_W]], axis=1)
    sl32 = jnp.concatenate([P[:, SOM_W:], z32], axis=1)
    mj0 = (iota_k % SOM_W != 0).astype(f32)
    mj31 = (iota_k % SOM_W != SOM_W - 1).astype(f32)
    pc = sr1 * mj0 + sl1 * mj31 + sr32 + sl32

    nv = (1.0 + (jj > 0).astype(f32) + (jj < SOM_W - 1).astype(f32)
          + (ii > 0).astype(f32) + (ii < SOM_H - 1).astype(f32))

    vals = (
        jnp.sum(m),
        jnp.sum(xn),
        jnp.sum(enc * pc) + jnp.sum(m),
        jnp.sum(nv * xn),
        jnp.sum(nv),
    )
    for k, v in enumerate(vals):
        prev = jnp.where(step == 0, 0.0, scal_ref[k])
        scal_ref[k] = prev + v


def kernel(inputs, embedding):
    x3 = inputs.reshape(2, C, M)
    et = embedding.T
    en = jnp.sum(embedding * embedding, axis=1)[None, :]
    ehi = et.astype(jnp.bfloat16)
    elo = (et - ehi.astype(jnp.float32)).astype(jnp.bfloat16)

    enc, qt, s = pl.pallas_call(
        _body,
        grid=(N // V,),
        in_specs=[
            pl.BlockSpec((1, C, V), lambda i: (i // (M // V), 0, i % (M // V))),
            pl.BlockSpec((C, K), lambda i: (0, 0)),
            pl.BlockSpec((1, K), lambda i: (0, 0)),
            pl.BlockSpec((C, K), lambda i: (0, 0)),
            pl.BlockSpec((C, K), lambda i: (0, 0)),
        ],
        out_specs=(
            pl.BlockSpec((V, K), lambda i: (i, 0)),
            pl.BlockSpec((1, C, V), lambda i: (i // (M // V), 0, i % (M // V))),
            pl.BlockSpec(memory_space=pltpu.SMEM),
        ),
        out_shape=(
            jax.ShapeDtypeStruct((N, K), jnp.float32),
            jax.ShapeDtypeStruct((2, C, M), jnp.float32),
            jax.ShapeDtypeStruct((8,), jnp.float32),
        ),
    )(x3, et, en, ehi, elo)

    commitment = (s[0] + s[1]) / (N * C)
    somloss = (s[2] + s[3]) / s[4]
    loss = ALPHA * commitment + BETA * somloss
    return (loss, qt.reshape(2, C, 8, 32, 32), enc)
